# Initial kernel scaffold; baseline (speedup 1.0000x reference)
import jax
import jax.numpy as jnp
from jax.experimental import pallas as pl


def kernel(node_features, edge_index, edge_features, action_mask, current_node, batch, W1, b1, W2, b2, W3, b3, Wa1, ba1, Wa2, ba2, Wc1, bc1, Wc2, bc2):
    raise NotImplementedError("write your pallas kernel here")


# speedup vs baseline: 4.6997x; 4.6997x over previous
import functools

import jax
import jax.numpy as jnp
from jax import lax
from jax.experimental import pallas as pl
from jax.experimental.pallas import tpu as pltpu

N, F, E = 10000, 128, 320000
B, A, H = 256, 10, 128


def _elu(x):
    return jnp.where(x > 0, x, jnp.exp(x) - 1.0)


def _fused_kernel(nf_ref, batch_ref, batchcol_ref, cur_ref, mask_ref,
                  w1_ref, b1_ref, w2_ref, b2_ref, w3_ref, b3_ref,
                  wa1_ref, ba1_ref, wa2_ref, ba2_ref,
                  wc1_ref, bc1_ref, wc2_ref, bc2_ref,
                  probs_ref, values_ref):
    f32 = jnp.float32

    x = nf_ref[...]
    h = _elu(jnp.dot(x, w1_ref[...], preferred_element_type=f32) + b1_ref[...])
    h = _elu(jnp.dot(h, w2_ref[...], preferred_element_type=f32) + b2_ref[...])
    h = _elu(jnp.dot(h, w3_ref[...], preferred_element_type=f32) + b3_ref[...])

    batch_row = batch_ref[...]

    h_scan = h
    b_col = batchcol_ref[...]
    s = 1
    while s < N:
        shifted = jnp.concatenate(
            [jnp.full((s, F), -jnp.inf, dtype=f32), h_scan[:N - s, :]], axis=0)
        bshift = jnp.concatenate(
            [jnp.full((s, 1), -1, dtype=jnp.int32), b_col[:N - s, :]], axis=0)
        same = (b_col == bshift)
        h_scan = jnp.where(same, jnp.maximum(h_scan, shifted), h_scan)
        s *= 2

    nxt = jnp.concatenate(
        [batch_row[:, 1:], jnp.full((1, 1), -1, dtype=jnp.int32)], axis=1)
    is_last = (batch_row != nxt).astype(f32)

    C = 2000
    seg_iota = lax.broadcasted_iota(jnp.int32, (B, C), 0)
    cur = cur_ref[...]
    seg_sum = jnp.zeros((B, F), dtype=f32)
    max_p = jnp.zeros((B, F), dtype=f32)
    current_emb = jnp.zeros((B, F), dtype=f32)
    counts = jnp.zeros((B, 1), dtype=f32)
    for off in range(0, N, C):
        b_chunk = batch_row[:, off:off + C]
        onehot = (seg_iota == b_chunk).astype(f32)
        counts = counts + jnp.sum(onehot, axis=1, keepdims=True)
        seg_sum = seg_sum + jnp.dot(onehot, h[off:off + C, :],
                                    preferred_element_type=f32)
        lastsel = onehot * is_last[:, off:off + C]
        max_p = max_p + jnp.dot(lastsel, h_scan[off:off + C, :],
                                preferred_element_type=f32)
        node_iota = lax.broadcasted_iota(jnp.int32, (B, C), 1) + off
        cur_onehot = (node_iota == cur).astype(f32)
        current_emb = current_emb + jnp.dot(cur_onehot, h[off:off + C, :],
                                            preferred_element_type=f32)
    mean_p = seg_sum / jnp.maximum(counts, 1.0)
    max_p = jnp.where(counts > 0, max_p, -jnp.inf)

    graph_emb = jnp.concatenate([mean_p, max_p], axis=-1)
    actor_in = jnp.concatenate([graph_emb, current_emb], axis=-1)
    a = _elu(jnp.dot(actor_in, wa1_ref[...], preferred_element_type=f32)
             + ba1_ref[...])
    logits = jnp.dot(a, wa2_ref[...], preferred_element_type=f32) + ba2_ref[...]

    amask = mask_ref[...]
    has_valid = jnp.sum(amask, axis=-1, keepdims=True) > 0
    safe_mask = jnp.where(has_valid, amask, jnp.ones_like(amask))
    logits = jnp.where(safe_mask == 0, -jnp.inf, logits)
    m = jnp.max(logits, axis=-1, keepdims=True)
    e = jnp.exp(logits - m)
    probs = e / jnp.sum(e, axis=-1, keepdims=True)
    nan_mask = jnp.any(jnp.isnan(probs), axis=-1, keepdims=True)
    probs = jnp.where(nan_mask, jnp.full_like(probs, 1.0 / A), probs)
    probs_ref[...] = probs

    c = _elu(jnp.dot(graph_emb, wc1_ref[...], preferred_element_type=f32)
             + bc1_ref[...])
    values_ref[...] = (jnp.dot(c, wc2_ref[...], preferred_element_type=f32)
                       + bc2_ref[...])


@jax.jit
def _run(node_features, action_mask, current_node, batch,
         W1, b1, W2, b2, W3, b3, Wa1, ba1, Wa2, ba2, Wc1, bc1, Wc2, bc2):
    batch_row = batch.astype(jnp.int32).reshape(1, N)
    batch_col = batch.astype(jnp.int32).reshape(N, 1)
    cur_col = current_node.astype(jnp.int32).reshape(B, 1)
    args = (node_features, batch_row, batch_col, cur_col, action_mask,
            W1, b1.reshape(1, H), W2, b2.reshape(1, H), W3, b3.reshape(1, H),
            Wa1, ba1.reshape(1, 256), Wa2, ba2.reshape(1, A),
            Wc1, bc1.reshape(1, 256), Wc2, bc2.reshape(1, 1))
    return pl.pallas_call(
        _fused_kernel,
        out_shape=(jax.ShapeDtypeStruct((B, A), jnp.float32),
                   jax.ShapeDtypeStruct((B, 1), jnp.float32)),
    )(*args)


def kernel(node_features, edge_index, edge_features, action_mask, current_node,
           batch, W1, b1, W2, b2, W3, b3, Wa1, ba1, Wa2, ba2, Wc1, bc1,
           Wc2, bc2):
    del edge_index, edge_features
    return _run(node_features, action_mask, current_node, batch,
                W1, b1, W2, b2, W3, b3, Wa1, ba1, Wa2, ba2,
                Wc1, bc1, Wc2, bc2)
